# Initial kernel scaffold; baseline (speedup 1.0000x reference)
import jax
import jax.numpy as jnp
from jax.experimental import pallas as pl


def kernel(x, edge_index, bn_gamma, bn_beta, emb_W, emb_b, W0, b0, W1, b1, W2, b2, ln0_g, ln0_b, ln1_g, ln1_b, ln2_g, ln2_b, fc1_W, fc1_b, fcn_g, fcn_b, fc2_W, fc2_b):
    raise NotImplementedError("write your pallas kernel here")


# speedup vs baseline: 8.1321x; 8.1321x over previous
import jax
import jax.numpy as jnp
from jax import lax
from jax.experimental import pallas as pl
from jax.experimental.pallas import tpu as pltpu
from jax.experimental.pallas import tpu_sc as plsc

N = 10000
E = 320000
D = 128
NC = 2
NS = 16
NW = NC * NS
CHUNK = 128
EP = E + (-E) % (NW * CHUNK)
ROWS = EP // CHUNK
RPW = ROWS // NW
NPAD = 10112
SLAB = NPAD // NS

_BN = float(1.0 / (1.0 + 1e-5) ** 0.5)


def _mesh():
    return plsc.VectorSubcoreMesh(
        core_axis_name="c", subcore_axis_name="s", num_cores=NC, num_subcores=NS
    )


def _sc_degree(dstI, zrows, onesD):
    def body(dst_h, z_h, ones_h, out_h, acc, didx, ones_v):
        c = lax.axis_index("c")
        s = lax.axis_index("s")
        wid = s * NC + c
        r0 = pl.multiple_of(s * SLAB, 8)
        pltpu.sync_copy(z_h.at[pl.ds(r0, SLAB)], acc.at[pl.ds(r0, SLAB)])
        pltpu.sync_copy(ones_h, ones_v)
        plsc.subcore_barrier()

        def chunk(j, carry):
            pltpu.sync_copy(dst_h.at[wid * RPW + j], didx)
            pltpu.sync_copy(ones_v, acc.at[didx], add=True)
            return carry

        lax.fori_loop(0, RPW, chunk, 0)
        plsc.subcore_barrier()
        pltpu.sync_copy(acc.at[pl.ds(r0, SLAB)], out_h.at[c, pl.ds(r0, SLAB)])

    return pl.kernel(
        body,
        out_type=jax.ShapeDtypeStruct((NC, NPAD, D), jnp.float32),
        mesh=_mesh(),
        scratch_types=[
            pltpu.VMEM_SHARED((NPAD, D), jnp.float32),
            pltpu.VMEM((CHUNK,), jnp.int32),
            pltpu.VMEM((CHUNK, D), jnp.float32),
        ],
    )(dstI, zrows, onesD)


def _sc_aggregate(g, zrows, srcI, dstI):
    def body(g_h, z_h, src_h, dst_h, out_h, acc, sidx, didx, rows, sem):
        c = lax.axis_index("c")
        s = lax.axis_index("s")
        wid = s * NC + c
        r0 = pl.multiple_of(s * SLAB, 8)
        pltpu.sync_copy(z_h.at[pl.ds(r0, SLAB)], acc.at[pl.ds(r0, SLAB)])
        plsc.subcore_barrier()

        def chunk(j, carry):
            row = wid * RPW + j
            pltpu.sync_copy(src_h.at[row], sidx)
            pltpu.sync_copy(dst_h.at[row], didx)
            pltpu.async_copy(g_h.at[sidx], rows, sem).wait()
            pltpu.sync_copy(rows, acc.at[didx], add=True)
            return carry

        lax.fori_loop(0, RPW, chunk, 0)
        plsc.subcore_barrier()
        pltpu.sync_copy(acc.at[pl.ds(r0, SLAB)], out_h.at[c, pl.ds(r0, SLAB)])

    return pl.kernel(
        body,
        out_type=jax.ShapeDtypeStruct((NC, NPAD, D), jnp.float32),
        mesh=_mesh(),
        scratch_types=[
            pltpu.VMEM_SHARED((NPAD, D), jnp.float32),
            pltpu.VMEM((CHUNK,), jnp.int32),
            pltpu.VMEM((CHUNK,), jnp.int32),
            pltpu.VMEM((CHUNK, D), jnp.float32),
            pltpu.SemaphoreType.DMA,
        ],
    )(g, zrows, srcI, dstI)


_B = 1000
_G = N // _B


def _full(shape):
    return pl.BlockSpec(shape, lambda i: tuple(0 for _ in shape))


def _tc_embed(x, deg16, gamma, beta, embW, embb, W0):
    def body(x_r, d_r, ga_r, be_r, ew_r, eb_r, w0_r, h_r, g_r, di_r):
        xb = x_r[...]
        hb = jnp.dot(xb * (ga_r[...] * _BN) + be_r[...], ew_r[...],
                     preferred_element_type=jnp.float32) + eb_r[...]
        dd = d_r[...]
        deg = jnp.sum(dd[0] + dd[1], axis=-1) * jnp.float32(1.0 / D) + 1.0
        dinv = lax.rsqrt(deg)[:, None]
        h_r[...] = hb
        g_r[...] = jnp.dot(hb, w0_r[...],
                           preferred_element_type=jnp.float32) * dinv
        di_r[...] = dinv

    return pl.pallas_call(
        body,
        grid=(_G,),
        in_specs=[
            pl.BlockSpec((_B, D), lambda i: (i, 0)),
            pl.BlockSpec((NC, _B, D), lambda i: (0, i, 0)),
            _full((1, D)), _full((1, D)), _full((D, D)), _full((1, D)),
            _full((D, D)),
        ],
        out_specs=[
            pl.BlockSpec((_B, D), lambda i: (i, 0)),
            pl.BlockSpec((_B, D), lambda i: (i, 0)),
            pl.BlockSpec((_B, 1), lambda i: (i, 0)),
        ],
        out_shape=[
            jax.ShapeDtypeStruct((N, D), jnp.float32),
            jax.ShapeDtypeStruct((N, D), jnp.float32),
            jax.ShapeDtypeStruct((N, 1), jnp.float32),
        ],
    )(x, deg16, gamma, beta, embW, embb, W0)


def _ln(z, g, b):
    mu = jnp.mean(z, axis=-1, keepdims=True)
    zc = z - mu
    var = jnp.mean(zc * zc, axis=-1, keepdims=True)
    return zc * lax.rsqrt(var + 1e-5) * g + b


def _leaky(z):
    return jnp.where(z >= 0, z, 0.1 * z)


def _tc_layer(a, gself, ident, dinv, b, lng, lnb, Wn):
    def body(a_r, gs_r, id_r, di_r, b_r, g_r, be_r, w_r, h_r, go_r):
        a2 = a_r[...]
        dcol = di_r[...]
        z = (a2[0] + a2[1] + gs_r[...]) * dcol + b_r[...]
        z = _leaky(_ln(z, g_r[...], be_r[...]))
        h1 = z + id_r[...]
        h_r[...] = h1
        go_r[...] = jnp.dot(h1, w_r[...],
                            preferred_element_type=jnp.float32) * dcol

    return pl.pallas_call(
        body,
        grid=(_G,),
        in_specs=[
            pl.BlockSpec((NC, _B, D), lambda i: (0, i, 0)),
            pl.BlockSpec((_B, D), lambda i: (i, 0)),
            pl.BlockSpec((_B, D), lambda i: (i, 0)),
            pl.BlockSpec((_B, 1), lambda i: (i, 0)),
            _full((1, D)), _full((1, D)), _full((1, D)), _full((D, D)),
        ],
        out_specs=[
            pl.BlockSpec((_B, D), lambda i: (i, 0)),
            pl.BlockSpec((_B, D), lambda i: (i, 0)),
        ],
        out_shape=[
            jax.ShapeDtypeStruct((N, D), jnp.float32),
            jax.ShapeDtypeStruct((N, D), jnp.float32),
        ],
    )(a, gself, ident, dinv, b, lng, lnb, Wn)


def _tc_final(a, gself, ident, dinv, b, lng, lnb, fc1W, fc1b, fcng, fcnb,
              fc2W, fc2b):
    def body(a_r, gs_r, id_r, di_r, b_r, g_r, be_r, f1w_r, f1b_r, fng_r, fnb_r,
             f2w_r, f2b_r, out_r, acc_r):
        i = pl.program_id(0)
        a2 = a_r[...]
        z = (a2[0] + a2[1] + gs_r[...]) * di_r[...] + b_r[...]
        z = _leaky(_ln(z, g_r[...], be_r[...]))
        h3 = z + id_r[...]
        part = jnp.sum(h3, axis=0, keepdims=True)

        @pl.when(i == 0)
        def _():
            acc_r[...] = part

        @pl.when(i > 0)
        def _():
            acc_r[...] = acc_r[...] + part

        @pl.when(i == _G - 1)
        def _():
            m = acc_r[...] * jnp.float32(1.0 / N)
            t = jnp.dot(m, f1w_r[...],
                        preferred_element_type=jnp.float32) + f1b_r[...]
            t = _leaky(_ln(t, fng_r[...], fnb_r[...]))
            out_r[...] = jnp.dot(t, f2w_r[...],
                                 preferred_element_type=jnp.float32) + f2b_r[...]

    return pl.pallas_call(
        body,
        grid=(_G,),
        in_specs=[
            pl.BlockSpec((NC, _B, D), lambda i: (0, i, 0)),
            pl.BlockSpec((_B, D), lambda i: (i, 0)),
            pl.BlockSpec((_B, D), lambda i: (i, 0)),
            pl.BlockSpec((_B, 1), lambda i: (i, 0)),
            _full((1, D)), _full((1, D)), _full((1, D)),
            _full((D, D)), _full((1, D)), _full((1, D)), _full((1, D)),
            _full((D, D)), _full((1, D)),
        ],
        out_specs=pl.BlockSpec((1, D), lambda i: (0, 0)),
        out_shape=jax.ShapeDtypeStruct((1, D), jnp.float32),
        scratch_shapes=[pltpu.VMEM((1, D), jnp.float32)],
    )(a, gself, ident, dinv, b, lng, lnb, fc1W, fc1b, fcng, fcnb, fc2W, fc2b)


def kernel(x, edge_index, bn_gamma, bn_beta, emb_W, emb_b, W0, b0, W1, b1,
           W2, b2, ln0_g, ln0_b, ln1_g, ln1_b, ln2_g, ln2_b, fc1_W, fc1_b,
           fcn_g, fcn_b, fc2_W, fc2_b):
    pad = EP - E
    srcI = jnp.pad(edge_index[0], (0, pad)).reshape(ROWS, CHUNK)
    dstI = jnp.pad(edge_index[1], (0, pad), constant_values=N).reshape(ROWS, CHUNK)
    onesD = jnp.ones((CHUNK, D), jnp.float32)
    zrows = jnp.zeros((NPAD, D), jnp.float32)
    r = lambda v: v.reshape(1, -1)

    deg16 = _sc_degree(dstI, zrows, onesD)
    h, g, dinv = _tc_embed(x, deg16, r(bn_gamma), r(bn_beta), emb_W,
                           r(emb_b), W0)
    a = _sc_aggregate(g, zrows, srcI, dstI)
    h, g = _tc_layer(a, g, h, dinv, r(b0), r(ln0_g), r(ln0_b), W1)
    a = _sc_aggregate(g, zrows, srcI, dstI)
    h, g = _tc_layer(a, g, h, dinv, r(b1), r(ln1_g), r(ln1_b), W2)
    a = _sc_aggregate(g, zrows, srcI, dstI)
    return _tc_final(a, g, h, dinv, r(b2), r(ln2_g), r(ln2_b), fc1_W, r(fc1_b),
                     r(fcn_g), r(fcn_b), fc2_W, r(fc2_b))
